# baseline (speedup 1.0000x reference)
import functools

import jax
import jax.numpy as jnp
from jax import lax
from jax.experimental import pallas as pl
from jax.experimental.pallas import tpu as pltpu
from jax.experimental.pallas import tpu_sc as plsc

_NUM_CORES = 2
_NUM_SUBCORES = 16
_NUM_WORKERS = _NUM_CORES * _NUM_SUBCORES


def _make_gather(embed, batch):
    b_per_w = batch // _NUM_WORKERS
    mesh = plsc.VectorSubcoreMesh(core_axis_name="c", subcore_axis_name="s")

    @functools.partial(
        pl.kernel,
        mesh=mesh,
        out_type=jax.ShapeDtypeStruct((batch, embed), jnp.float32),
        scratch_types=[
            pltpu.VMEM((b_per_w,), jnp.int32),
            pltpu.VMEM((b_per_w, embed), jnp.float32),
            pltpu.SemaphoreType.DMA,
        ],
        compiler_params=pltpu.CompilerParams(use_tc_tiling_on_sc=False),
    )
    def gather_kernel(table_hbm, idx_hbm, out_hbm, idx_v, rows_v, sem):
        wid = lax.axis_index("s") * _NUM_CORES + lax.axis_index("c")
        base = wid * b_per_w
        pltpu.sync_copy(idx_hbm.at[pl.ds(base, b_per_w)], idx_v)
        pltpu.async_copy(table_hbm.at[idx_v], rows_v, sem).wait()
        pltpu.sync_copy(rows_v, out_hbm.at[pl.ds(base, b_per_w)])

    return gather_kernel


def kernel(node_embs, node_ids):
    _, embed = node_embs.shape
    (batch,) = node_ids.shape
    gather = _make_gather(embed, batch)
    return gather(node_embs, node_ids.astype(jnp.int32))


# speedup vs baseline: 1.7175x; 1.7175x over previous
import functools

import jax
import jax.numpy as jnp
from jax import lax
from jax.experimental import pallas as pl
from jax.experimental.pallas import tpu as pltpu
from jax.experimental.pallas import tpu_sc as plsc

_NUM_CORES = 2
_NUM_SUBCORES = 16
_NUM_WORKERS = _NUM_CORES * _NUM_SUBCORES
_LANES = 16


def _make_gather(embed, batch):
    b_per_w = batch // _NUM_WORKERS
    mesh = plsc.VectorSubcoreMesh(core_axis_name="c", subcore_axis_name="s")

    @functools.partial(
        pl.kernel,
        mesh=mesh,
        out_type=jax.ShapeDtypeStruct((batch, embed), jnp.float32),
        scratch_types=[
            pltpu.VMEM((b_per_w,), jnp.int32),
            pltpu.VMEM((b_per_w, embed), jnp.float32),
            pltpu.SemaphoreType.DMA,
        ],
    )
    def gather_kernel(table_hbm, idx_hbm, out_hbm, ids_v, out_v, sem):
        wid = lax.axis_index("s") * _NUM_CORES + lax.axis_index("c")
        base = wid * b_per_w
        pltpu.sync_copy(idx_hbm.at[pl.ds(base, b_per_w)], ids_v)

        def row_body(g, _):
            ids16 = ids_v[pl.ds(g * _LANES, _LANES)]
            for k in range(_LANES):
                pltpu.async_copy(
                    table_hbm.at[ids16[k]],
                    out_v.at[g * _LANES + k],
                    sem,
                )
            return 0

        lax.fori_loop(0, b_per_w // _LANES, row_body, 0)
        pltpu.make_async_copy(
            table_hbm.at[pl.ds(0, b_per_w)], out_v, sem
        ).wait()
        pltpu.sync_copy(out_v, out_hbm.at[pl.ds(base, b_per_w)])

    return gather_kernel


def kernel(node_embs, node_ids):
    _, embed = node_embs.shape
    (batch,) = node_ids.shape
    gather = _make_gather(embed, batch)
    return gather(node_embs, node_ids.astype(jnp.int32))


# speedup vs baseline: 1.7236x; 1.0036x over previous
import functools

import jax
import jax.numpy as jnp
from jax import lax
from jax.experimental import pallas as pl
from jax.experimental.pallas import tpu as pltpu
from jax.experimental.pallas import tpu_sc as plsc

_NUM_CORES = 2
_NUM_SUBCORES = 16
_NUM_WORKERS = _NUM_CORES * _NUM_SUBCORES
_LANES = 16


def _make_gather(embed, batch):
    b_per_w = batch // _NUM_WORKERS
    mesh = plsc.VectorSubcoreMesh(core_axis_name="c", subcore_axis_name="s")

    @functools.partial(
        pl.kernel,
        mesh=mesh,
        out_type=jax.ShapeDtypeStruct((batch, embed), jnp.float32),
        scratch_types=[
            pltpu.VMEM((b_per_w,), jnp.int32),
            pltpu.VMEM((b_per_w, embed), jnp.float32),
            pltpu.SemaphoreType.DMA,
            pltpu.SemaphoreType.DMA,
            pltpu.SemaphoreType.DMA,
            pltpu.SemaphoreType.DMA,
        ],
    )
    def gather_kernel(table_hbm, idx_hbm, out_hbm, ids_v, out_v,
                      sem0, sem1, sem2, sem3):
        sems = (sem0, sem1, sem2, sem3)
        wid = lax.axis_index("s") * _NUM_CORES + lax.axis_index("c")
        base = wid * b_per_w
        pltpu.sync_copy(idx_hbm.at[pl.ds(base, b_per_w)], ids_v)

        def row_body(g, _):
            ids16 = ids_v[pl.ds(g * _LANES, _LANES)]
            for k in range(_LANES):
                pltpu.async_copy(
                    table_hbm.at[ids16[k]],
                    out_v.at[g * _LANES + k],
                    sems[k % 4],
                )
            return 0

        lax.fori_loop(0, b_per_w // _LANES, row_body, 0)
        for s in sems:
            pltpu.make_async_copy(
                table_hbm.at[pl.ds(0, b_per_w // 4)],
                out_v.at[pl.ds(0, b_per_w // 4)],
                s,
            ).wait()
        pltpu.sync_copy(out_v, out_hbm.at[pl.ds(base, b_per_w)])

    return gather_kernel


def kernel(node_embs, node_ids):
    _, embed = node_embs.shape
    (batch,) = node_ids.shape
    gather = _make_gather(embed, batch)
    return gather(node_embs, node_ids.astype(jnp.int32))
